# baseline (speedup 1.0000x reference)
import jax
import jax.numpy as jnp
from jax.experimental import pallas as pl
from jax.experimental.pallas import tpu as pltpu


def _add_pos_kernel(x_ref, h_ref, w_ref, o_ref, s_ref):
    @pl.when(pl.program_id(0) == 0)
    def _():
        h = h_ref[...][:, :, None, :]
        w = w_ref[...][:, None, :, :]
        s_ref[...] = h + w

    o_ref[...] = x_ref[...] + s_ref[...]


def kernel(x, h_table, w_table):
    B, H, W, D = x.shape
    return pl.pallas_call(
        _add_pos_kernel,
        grid=(B,),
        in_specs=[
            pl.BlockSpec((1, H, W, D), lambda b: (b, 0, 0, 0)),
            pl.BlockSpec((1, H, D), lambda b: (0, 0, 0)),
            pl.BlockSpec((1, W, D), lambda b: (0, 0, 0)),
        ],
        out_specs=pl.BlockSpec((1, H, W, D), lambda b: (b, 0, 0, 0)),
        out_shape=jax.ShapeDtypeStruct((B, H, W, D), x.dtype),
        scratch_shapes=[pltpu.VMEM((1, H, W, D), x.dtype)],
    )(x, h_table[None], w_table[None])
